# baseline (speedup 1.0000x reference)
import functools

import jax
import jax.numpy as jnp
import numpy as np
from jax.experimental import pallas as pl
from jax.experimental.pallas import tpu as pltpu

F1 = 8
D = 16
K = 5
T = 640
CH = 64
NC = 4
B = 32
KW1 = 80
PAD1 = 39
KW3 = 16
PAD3 = 7
TB = 128
NBLK = T // TB
WIN = TB + KW1 - 1
T2 = T // 4
T3 = T2 // 8


def _elu(x):
    return jnp.where(x > 0, x, jnp.exp(x) - 1.0)


def _make_band_sel():
    i = np.arange(WIN)[:, None]
    j = np.arange(TB)[None, :]
    diff = (i - j)[None, :, :]
    tau = np.arange(KW1)[:, None, None]
    return (diff == tau).astype(np.float32).reshape(KW1, WIN * TB)


_BAND_SEL = jnp.asarray(_make_band_sel())


def _body(x_ref, l_ref, w5_ref, band_ref, wsep_ref, p4_ref, fcw_ref,
          s1_ref, e1_ref, b1_ref, chb_ref, s2_ref, c2_ref, a3_ref, d3_ref,
          fcb_ref, out_ref):
    f32 = jnp.float32

    hi = jax.lax.Precision.HIGHEST
    Lm = l_ref[...]
    v0 = jnp.full((1, CH), 1.0 / CH, f32)
    v1 = jnp.dot(v0, Lm, preferred_element_type=f32, precision=hi)
    v2 = 2.0 * jnp.dot(v1, Lm, preferred_element_type=f32, precision=hi) - v0
    v3 = 2.0 * jnp.dot(v2, Lm, preferred_element_type=f32, precision=hi) - v1
    v4 = 2.0 * jnp.dot(v3, Lm, preferred_element_type=f32, precision=hi) - v2
    V = jnp.concatenate([v0, v1, v2, v3, v4], axis=0)
    A = jnp.dot(V.T, w5_ref[...], preferred_element_type=f32,
                precision=hi)

    sumA = jnp.sum(A, axis=0, keepdims=True).T
    sV = jnp.sum(V, axis=1, keepdims=True)
    sW = jnp.sum(sV * w5_ref[...], axis=0, keepdims=True).T
    chb = chb_ref[...].reshape(D, 1)
    e1 = e1_ref[...]
    b1 = b1_ref[...]
    s1 = s1_ref[...]
    s2 = s2_ref[...]
    c2 = c2_ref[...]
    bias2 = e1 * sumA + b1 * sW + chb
    alpha = s1 * s2
    beta = s2 * bias2 + c2

    bf16 = jnp.bfloat16
    X = x_ref[...]
    Y = jnp.einsum('bct,cd->bdt', X, A,
                   preferred_element_type=f32)
    Yr = Y.reshape(B * D, T).astype(bf16)
    z = jnp.zeros((B * D, PAD1), bf16)
    z2 = jnp.zeros((B * D, KW1 - 1 - PAD1), bf16)
    Ypad = jnp.concatenate([z, Yr, z2], axis=1)

    band = band_ref[...]
    p4 = p4_ref[...]
    wins = jnp.concatenate(
        [Ypad[:, blk * TB: blk * TB + WIN] for blk in range(NBLK)],
        axis=0)
    zb = jnp.dot(wins, band, preferred_element_type=f32)
    z5 = zb.reshape(NBLK, B, D, F1, TB)
    e = _elu(z5 * alpha[None, None, :, :, None]
             + beta[None, None, :, :, None])
    e2 = e.reshape(NBLK * B * D * F1, TB).astype(bf16)
    ep = jnp.dot(e2, p4, preferred_element_type=f32)
    ep5 = ep.reshape(NBLK, B, D, F1, TB // 4).astype(bf16)
    P = jnp.transpose(ep5, (1, 0, 4, 3, 2)).reshape(B, T2, F1 * D)

    zp3 = jnp.zeros((B, PAD3, F1 * D), bf16)
    zp4 = jnp.zeros((B, KW3 - 1 - PAD3, F1 * D), bf16)
    Ppad = jnp.concatenate([zp3, P, zp4], axis=1)
    wsep = wsep_ref[...].reshape(KW3, F1 * D, F1 * D)
    acc = jnp.zeros((B, T2, F1 * D), f32)
    for tau in range(KW3):
        acc = acc + jax.lax.dot_general(
            Ppad[:, tau: tau + T2, :], wsep[tau],
            (((2,), (0,)), ((), ())),
            preferred_element_type=f32)
    acc = acc.reshape(B * T2, F1 * D)

    S = _elu(acc * a3_ref[...] + d3_ref[...])
    Q = S.reshape(B, T3, 8, F1 * D).mean(axis=2)
    Qf = Q.reshape(B, T3 * F1 * D)
    out = jnp.dot(Qf, fcw_ref[...], preferred_element_type=f32)
    out_ref[...] = out + fcb_ref[...]


@functools.partial(jax.jit, static_argnums=())
def kernel(X, L, conv1_w, conv1_b, bn1_g, bn1_b, cheb_W, cheb_b, bn2_g,
           bn2_b, sep_w, sep_b, bn3_g, bn3_b, fc_w, fc_b):
    f32 = jnp.float32
    rs = 1.0 / jnp.sqrt(jnp.float32(1.0 + 1e-5))

    s1 = (bn1_g * rs).reshape(1, F1)
    e1 = (s1 * conv1_b.reshape(1, F1))
    b1 = bn1_b.reshape(1, F1)
    chb = cheb_b.reshape(1, D)
    s2 = (bn2_g * rs).reshape(F1, D).T
    c2 = bn2_b.reshape(F1, D).T
    a3 = (bn3_g * rs).reshape(1, F1 * D)
    d3 = (a3 * sep_b.reshape(1, F1 * D) + bn3_b.reshape(1, F1 * D))

    w1 = conv1_w.reshape(F1, KW1)
    band = jnp.transpose(
        jnp.dot(w1, _BAND_SEL, preferred_element_type=f32).reshape(
            F1, WIN, TB),
        (1, 0, 2)).reshape(WIN, F1 * TB).astype(jnp.bfloat16)

    p4 = (jnp.kron(jnp.eye(TB // 4, dtype=f32), jnp.ones((4, 1), f32))
          * 0.25).astype(jnp.bfloat16)

    wsep = jnp.transpose(sep_w[:, :, 0, :], (2, 1, 0)).reshape(
        KW3 * F1 * D, F1 * D).astype(jnp.bfloat16)

    fcw = jnp.transpose(fc_w.reshape(NC, F1 * D, T3), (2, 1, 0)).reshape(
        T3 * F1 * D, NC)
    fcb = fc_b.reshape(1, NC)

    w5 = cheb_W[:, 0, :]

    return pl.pallas_call(
        _body,
        out_shape=jax.ShapeDtypeStruct((B, NC), f32),
        compiler_params=pltpu.CompilerParams(
            vmem_limit_bytes=100 * 1024 * 1024),
    )(X.astype(f32), L[0], w5, band, wsep, p4, fcw,
      s1, e1, b1, chb, s2, c2, a3, d3, fcb)


# speedup vs baseline: 1.1292x; 1.1292x over previous
import functools

import jax
import jax.numpy as jnp
import numpy as np
from jax.experimental import pallas as pl
from jax.experimental.pallas import tpu as pltpu

F1 = 8
D = 16
K = 5
T = 640
CH = 64
NC = 4
B = 32
KW1 = 80
PAD1 = 39
KW3 = 16
PAD3 = 7
TB = 128
NBLK = T // TB
WIN = TB + KW1 - 1
T2 = T // 4
T3 = T2 // 8


def _elu(x):
    return jnp.where(x > 0, x, jnp.exp(x) - 1.0)


def _make_band_sel():
    i = np.arange(WIN)[:, None]
    j = np.arange(TB)[None, :]
    diff = (i - j)[None, :, :]
    tau = np.arange(KW1)[:, None, None]
    return (diff == tau).astype(np.float32).reshape(KW1, WIN * TB)


_BAND_SEL = jnp.asarray(_make_band_sel())


def _body(x_ref, l_ref, w5_ref, band_ref, wsep_ref, p4_ref, fcw_ref,
          bn1g_ref, c1b_ref, bn1b_ref, chb0_ref, bn2g_ref, bn2b_ref,
          bn3g_ref, sepb_ref, bn3b_ref, fcb_ref, out_ref):
    f32 = jnp.float32
    rs = 0.9999950000374997

    hi = jax.lax.Precision.HIGHEST
    Lm = l_ref[...]
    v0 = jnp.full((1, CH), 1.0 / CH, f32)
    v1 = jnp.dot(v0, Lm, preferred_element_type=f32, precision=hi)
    v2 = 2.0 * jnp.dot(v1, Lm, preferred_element_type=f32, precision=hi) - v0
    v3 = 2.0 * jnp.dot(v2, Lm, preferred_element_type=f32, precision=hi) - v1
    v4 = 2.0 * jnp.dot(v3, Lm, preferred_element_type=f32, precision=hi) - v2
    V = jnp.concatenate([v0, v1, v2, v3, v4], axis=0)
    A = jnp.dot(V.T, w5_ref[...], preferred_element_type=f32,
                precision=hi)

    sumA = jnp.sum(A, axis=0, keepdims=True).T
    sV = jnp.sum(V, axis=1, keepdims=True)
    sW = jnp.sum(sV * w5_ref[...], axis=0, keepdims=True).T
    chb = chb0_ref[...].reshape(D, 1)
    s1 = bn1g_ref[...] * rs
    e1 = s1 * c1b_ref[...]
    b1 = bn1b_ref[...]
    s2 = bn2g_ref[...].T * rs
    c2 = bn2b_ref[...].T
    bias2 = e1 * sumA + b1 * sW + chb
    alpha = s1 * s2
    beta = s2 * bias2 + c2
    a3 = bn3g_ref[...] * rs
    d3 = a3 * sepb_ref[...] + bn3b_ref[...]

    bf16 = jnp.bfloat16
    X = x_ref[...]
    Y = jnp.einsum('bct,cd->bdt', X, A,
                   preferred_element_type=f32)
    Yr = Y.reshape(B * D, T).astype(bf16)
    z = jnp.zeros((B * D, PAD1), bf16)
    z2 = jnp.zeros((B * D, KW1 - 1 - PAD1), bf16)
    Ypad = jnp.concatenate([z, Yr, z2], axis=1)

    band = band_ref[...]
    p4 = p4_ref[...]
    wins = jnp.concatenate(
        [Ypad[:, blk * TB: blk * TB + WIN] for blk in range(NBLK)],
        axis=0)
    zb = jnp.dot(wins, band, preferred_element_type=f32)
    z5 = zb.reshape(NBLK, B, D, F1, TB)
    e = _elu(z5 * alpha[None, None, :, :, None]
             + beta[None, None, :, :, None])
    e2 = e.reshape(NBLK * B * D * F1, TB).astype(bf16)
    ep = jnp.dot(e2, p4, preferred_element_type=f32)
    ep5 = ep.reshape(NBLK, B, D, F1, TB // 4).astype(bf16)
    P = jnp.transpose(ep5, (1, 0, 4, 3, 2)).reshape(B, T2, F1 * D)

    zp3 = jnp.zeros((B, PAD3, F1 * D), bf16)
    zp4 = jnp.zeros((B, KW3 - 1 - PAD3, F1 * D), bf16)
    Ppad = jnp.concatenate([zp3, P, zp4], axis=1)
    wsep = wsep_ref[...]
    Pim = jnp.concatenate(
        [Ppad[:, tau: tau + T2, :] for tau in range(KW3)],
        axis=2)
    acc = jnp.dot(Pim.reshape(B * T2, KW3 * F1 * D), wsep,
                  preferred_element_type=f32)

    S = _elu(acc * a3 + d3)
    Q = S.reshape(B, T3, 8, F1 * D).mean(axis=2)
    Qf = Q.reshape(B, T3 * F1 * D)
    out = jnp.dot(Qf, fcw_ref[...], preferred_element_type=f32)
    out_ref[...] = out + fcb_ref[...]


@functools.partial(jax.jit, static_argnums=())
def kernel(X, L, conv1_w, conv1_b, bn1_g, bn1_b, cheb_W, cheb_b, bn2_g,
           bn2_b, sep_w, sep_b, bn3_g, bn3_b, fc_w, fc_b):
    f32 = jnp.float32

    w1 = conv1_w.reshape(F1, KW1)
    band = jnp.transpose(
        jnp.dot(w1, _BAND_SEL, preferred_element_type=f32).reshape(
            F1, WIN, TB),
        (1, 0, 2)).reshape(WIN, F1 * TB).astype(jnp.bfloat16)

    p4 = (jnp.kron(jnp.eye(TB // 4, dtype=f32), jnp.ones((4, 1), f32))
          * 0.25).astype(jnp.bfloat16)

    wsep = jnp.transpose(sep_w[:, :, 0, :], (2, 1, 0)).reshape(
        KW3 * F1 * D, F1 * D).astype(jnp.bfloat16)

    fcw = jnp.transpose(fc_w.reshape(NC, F1 * D, T3), (2, 1, 0)).reshape(
        T3 * F1 * D, NC)
    fcb = fc_b.reshape(1, NC)

    w5 = cheb_W.reshape(K, D)

    return pl.pallas_call(
        _body,
        out_shape=jax.ShapeDtypeStruct((B, NC), f32),
        compiler_params=pltpu.CompilerParams(
            vmem_limit_bytes=100 * 1024 * 1024),
    )(X.astype(f32), L.reshape(CH, CH), w5, band, wsep, p4, fcw,
      bn1_g.reshape(1, F1), conv1_b.reshape(1, F1), bn1_b.reshape(1, F1),
      cheb_b.reshape(1, D), bn2_g.reshape(F1, D), bn2_b.reshape(F1, D),
      bn3_g.reshape(1, F1 * D), sep_b.reshape(1, F1 * D),
      bn3_b.reshape(1, F1 * D), fc_b.reshape(1, NC))
